# baseline (speedup 1.0000x reference)
import jax
import jax.numpy as jnp
from jax import lax
from jax.experimental import pallas as pl
from jax.experimental.pallas import tpu as pltpu

_N = 262144
_C = 128
_B = 16384
_G = _N // _B
_R = _B // _C
_IGNORE = 0


def _fused_kernel(x_ref, t_ref, out_ref, acc_ref, npos_ref):
    pb = pl.program_id(0)
    x = x_ref[...]
    t_p = t_ref[...]

    e = jnp.exp(x)
    s_col = jnp.sum(e, axis=1, keepdims=True)
    lse_col = jnp.log(s_col)
    lse_p = lse_col.reshape(_R, _C)

    x3 = x.reshape(_R, _C, _C)
    t3 = t_p.reshape(_R, 1, _C)
    _CH = 8
    q_iota = lax.broadcasted_iota(jnp.int32, (_CH, _C, _C), 1)
    l_iota = lax.broadcasted_iota(jnp.int32, (_CH, _C, _C), 2)
    eye3 = q_iota == l_iota
    xt_chunks = []
    for c in range(_R // _CH):
        x3c = lax.slice_in_dim(x3, c * _CH, (c + 1) * _CH, axis=0)
        t3c = jnp.broadcast_to(
            lax.slice_in_dim(t3, c * _CH, (c + 1) * _CH, axis=0),
            (_CH, _C, _C))
        g3c = jnp.take_along_axis(x3c, t3c, axis=2)
        xt_chunks.append(jnp.sum(jnp.where(eye3, g3c, 0.0), axis=1))
    xt_p = jnp.concatenate(xt_chunks, axis=0)

    valid_p = t_p != _IGNORE
    acc_ref[pl.ds(pb * _R, _R), :] = jnp.where(valid_p, lse_p - xt_p, 0.0)

    nv = jnp.sum(valid_p.astype(jnp.int32))
    npos_ref[0] = jnp.where(pb == 0, nv, npos_ref[0] + nv)

    @pl.when(pb == _G - 1)
    def _select():
        loss = acc_ref[...]
        bits = lax.bitcast_convert_type(loss, jnp.int32)
        p = npos_ref[0]
        k = jnp.minimum(3 * p, _N - p)

        def body(_, carry):
            lo, hi = carry
            t2 = hi - (hi - lo) // 2
            t3 = hi - (hi - t2) // 2
            t1 = jnp.maximum(lo, (t2 - 1) - (t2 - 1 - lo) // 2)
            c1 = jnp.sum((bits >= t1).astype(jnp.int32))
            c2 = jnp.sum((bits >= t2).astype(jnp.int32))
            c3 = jnp.sum((bits >= t3).astype(jnp.int32))
            ok1, ok2, ok3 = c1 >= k, c2 >= k, c3 >= k
            lo2 = jnp.where(ok2, jnp.where(ok3, t3, t2),
                            jnp.where(ok1, t1, lo))
            hi2 = jnp.where(ok2, jnp.where(ok3, hi, t3 - 1),
                            jnp.where(ok1, t2 - 1, t1 - 1))
            return lo2, hi2

        lo, _ = lax.fori_loop(0, 16, body,
                              (jnp.int32(0), jnp.int32(2**31 - 1)))
        v = lax.bitcast_convert_type(lo, jnp.float32)
        gt = bits > lo
        cnt_gt = jnp.sum(gt.astype(jnp.int32))
        sum_gt = jnp.sum(jnp.where(gt, loss, 0.0))
        kf = k.astype(jnp.float32)
        out_ref[0, 0] = (sum_gt + (kf - cnt_gt.astype(jnp.float32)) * v) / kf


def kernel(input, target):
    t_p = target.astype(jnp.int32).reshape(_N // _C, _C)
    out = pl.pallas_call(
        _fused_kernel,
        grid=(_G,),
        in_specs=[
            pl.BlockSpec((_B, _C), lambda i: (i, 0)),
            pl.BlockSpec((_R, _C), lambda i: (i, 0)),
        ],
        out_specs=pl.BlockSpec(memory_space=pltpu.SMEM),
        out_shape=jax.ShapeDtypeStruct((1, 1), jnp.float32),
        scratch_shapes=[
            pltpu.VMEM((_N // _C, _C), jnp.float32),
            pltpu.SMEM((1,), jnp.int32),
        ],
    )(input, t_p)
    return out[0, 0]


# speedup vs baseline: 1.0006x; 1.0006x over previous
import jax
import jax.numpy as jnp
from jax import lax
from jax.experimental import pallas as pl
from jax.experimental.pallas import tpu as pltpu

_N = 262144
_C = 128
_B = 16384
_G = _N // _B
_R = _B // _C
_IGNORE = 0


def _fused_kernel(x_ref, t_ref, out_ref, acc_ref, npos_ref):
    pb = pl.program_id(0)
    x = x_ref[...]
    t_p = t_ref[...]

    e = jnp.exp(x)
    s_col = jnp.sum(e, axis=1, keepdims=True)
    lse_col = jnp.log(s_col)
    lse_p = lse_col.reshape(_R, _C)

    x3 = x.reshape(_R, _C, _C)
    t3 = jnp.broadcast_to(t_p.reshape(_R, 1, _C), (_R, _C, _C))
    g3 = jnp.take_along_axis(x3, t3, axis=2)
    q_iota = lax.broadcasted_iota(jnp.int32, (_R, _C, _C), 1)
    l_iota = lax.broadcasted_iota(jnp.int32, (_R, _C, _C), 2)
    xt_p = jnp.sum(jnp.where(q_iota == l_iota, g3, 0.0), axis=1)

    valid_p = t_p != _IGNORE
    acc_ref[pl.ds(pb * _R, _R), :] = jnp.where(valid_p, lse_p - xt_p, 0.0)

    nv = jnp.sum(valid_p.astype(jnp.int32))
    npos_ref[0] = jnp.where(pb == 0, nv, npos_ref[0] + nv)

    @pl.when(pb == _G - 1)
    def _select():
        loss = acc_ref[...]
        bits = lax.bitcast_convert_type(loss, jnp.int32)
        p = npos_ref[0]
        k = jnp.minimum(3 * p, _N - p)

        def body(_, carry):
            lo, hi = carry
            t2 = hi - (hi - lo) // 2
            t3 = hi - (hi - t2) // 2
            t1 = jnp.maximum(lo, (t2 - 1) - (t2 - 1 - lo) // 2)
            c1 = jnp.sum((bits >= t1).astype(jnp.int32))
            c2 = jnp.sum((bits >= t2).astype(jnp.int32))
            c3 = jnp.sum((bits >= t3).astype(jnp.int32))
            ok1, ok2, ok3 = c1 >= k, c2 >= k, c3 >= k
            lo2 = jnp.where(ok2, jnp.where(ok3, t3, t2),
                            jnp.where(ok1, t1, lo))
            hi2 = jnp.where(ok2, jnp.where(ok3, hi, t3 - 1),
                            jnp.where(ok1, t2 - 1, t1 - 1))
            return lo2, hi2

        lo, _ = lax.fori_loop(0, 16, body,
                              (jnp.int32(0), jnp.int32(2**31 - 1)))
        v = lax.bitcast_convert_type(lo, jnp.float32)
        gt = bits > lo
        cnt_gt = jnp.sum(gt.astype(jnp.int32))
        sum_gt = jnp.sum(jnp.where(gt, loss, 0.0))
        kf = k.astype(jnp.float32)
        out_ref[0, 0] = (sum_gt + (kf - cnt_gt.astype(jnp.float32)) * v) / kf


def kernel(input, target):
    t_p = target.astype(jnp.int32).reshape(_N // _C, _C)
    out = pl.pallas_call(
        _fused_kernel,
        grid=(_G,),
        in_specs=[
            pl.BlockSpec((_B, _C), lambda i: (i, 0)),
            pl.BlockSpec((_R, _C), lambda i: (i, 0)),
        ],
        out_specs=pl.BlockSpec(memory_space=pltpu.SMEM),
        out_shape=jax.ShapeDtypeStruct((1, 1), jnp.float32),
        scratch_shapes=[
            pltpu.VMEM((_N // _C, _C), jnp.float32),
            pltpu.SMEM((1,), jnp.int32),
        ],
    )(input, t_p)
    return out[0, 0]
